# Initial kernel scaffold; baseline (speedup 1.0000x reference)
import jax
import jax.numpy as jnp
from jax.experimental import pallas as pl


def kernel(x, lens_effects, diskernel, lens_mask):
    raise NotImplementedError("write your pallas kernel here")


# speedup vs baseline: 38.3831x; 38.3831x over previous
import functools

import numpy as np
import jax
import jax.numpy as jnp
from jax.experimental import pallas as pl
from jax.experimental.pallas import tpu as pltpu

_L = 7
_R = _L // 2
_H = 512
_W = 512
_TH = 128


def _disc_offsets():
    offs = []
    for ey in range(-_R, _R + 1):
        for ex in range(-_R, _R + 1):
            if ey * ey + ex * ex <= _R * _R:
                offs.append((ey, ex))
    return offs

_OFFSETS = _disc_offsets()


def _scatter_body(le_ref, dk_ref, mask_ref, x_ref, o_ref, pad_ref):
    le = le_ref[pl.program_id(0), 0]
    rad = jnp.abs(x_ref[0, 3]) * le

    pad_ref[...] = jnp.zeros(pad_ref.shape, jnp.float32)
    pad_ref[0, _R:_R + _H, _R:_R + _W] = x_ref[0, 0]
    pad_ref[1, _R:_R + _H, _R:_R + _W] = x_ref[0, 1]
    pad_ref[2, _R:_R + _H, _R:_R + _W] = x_ref[0, 2]
    pad_ref[3, _R:_R + _H, _R:_R + _W] = rad

    for y0 in range(0, _H, _TH):
        acc_r = None
        acc_g = None
        acc_b = None
        acc_w = None
        for (ey, ex) in _OFFSETS:
            sy = _R + ey + y0
            sx = _R + ex
            d = dk_ref[_R - ey, _R - ex]
            m = mask_ref[_R - ey, _R - ex]
            srad = pad_ref[3, sy:sy + _TH, sx:sx + _W]
            w = jnp.clip(srad - (d - 1.0), 0.0, 1.0) * m
            cr = w * pad_ref[0, sy:sy + _TH, sx:sx + _W]
            cg = w * pad_ref[1, sy:sy + _TH, sx:sx + _W]
            cb = w * pad_ref[2, sy:sy + _TH, sx:sx + _W]
            if acc_w is None:
                acc_w, acc_r, acc_g, acc_b = w, cr, cg, cb
            else:
                acc_w += w
                acc_r += cr
                acc_g += cg
                acc_b += cb

        inv = 1.0 / (acc_w + 1e-8)
        o_ref[0, 0, y0:y0 + _TH, :] = acc_r * inv
        o_ref[0, 1, y0:y0 + _TH, :] = acc_g * inv
        o_ref[0, 2, y0:y0 + _TH, :] = acc_b * inv


@jax.jit
def kernel(x, lens_effects, diskernel, lens_mask):
    b, c, h, w = x.shape
    out = pl.pallas_call(
        _scatter_body,
        grid=(b,),
        in_specs=[
            pl.BlockSpec((b, 1), lambda i: (0, 0), memory_space=pltpu.SMEM),
            pl.BlockSpec((_L, _L), lambda i: (0, 0), memory_space=pltpu.SMEM),
            pl.BlockSpec((_L, _L), lambda i: (0, 0), memory_space=pltpu.SMEM),
            pl.BlockSpec((1, 4, h, w), lambda i: (i, 0, 0, 0)),
        ],
        out_specs=pl.BlockSpec((1, 3, h, w), lambda i: (i, 0, 0, 0)),
        out_shape=jax.ShapeDtypeStruct((b, 3, h, w), x.dtype),
        scratch_shapes=[pltpu.VMEM((4, h + 2 * _R, w + 2 * _R), jnp.float32)],
    )(lens_effects, diskernel, lens_mask, x)
    return out


# speedup vs baseline: 64.3127x; 1.6755x over previous
import numpy as np
import jax
import jax.numpy as jnp
from jax.experimental import pallas as pl
from jax.experimental.pallas import tpu as pltpu

_L = 7
_R = _L // 2
_H = 512
_W = 512
_TH = 128
_PW = _W + 2 * _R


def _rings():
    rings = {}
    for ey in range(-_R, _R + 1):
        for ex in range(-_R, _R + 1):
            d2 = ey * ey + ex * ex
            if d2 <= _R * _R:
                rings.setdefault(d2, []).append((ey, ex))
    out = []
    for d2, offs in sorted(rings.items()):
        groups = {}
        for (ey, ex) in offs:
            groups.setdefault(abs(ey), set()).add(ex)
        out.append((d2, sorted((ady, sorted(dxs)) for ady, dxs in groups.items())))
    return out

_RINGS = _rings()


def _scatter_body(le_ref, dk_ref, mask_ref, x_ref, o_ref, pad_ref):
    le = le_ref[pl.program_id(0), 0]
    rad = jnp.abs(x_ref[0, 3]) * le

    pad_ref[...] = jnp.zeros(pad_ref.shape, jnp.float32)
    pad_ref[0, _R:_R + _H, _R:_R + _W] = x_ref[0, 0]
    pad_ref[1, _R:_R + _H, _R:_R + _W] = x_ref[0, 1]
    pad_ref[2, _R:_R + _H, _R:_R + _W] = x_ref[0, 2]
    pad_ref[3, _R:_R + _H, _R:_R + _W] = rad

    for y0 in range(0, _H, _TH):
        acc = [None, None, None, None]
        prad = pad_ref[3, y0:y0 + _TH + 2 * _R, :]
        for d2, groups in _RINGS:
            rey, rex = next((ey, ex) for ey in range(-_R, _R + 1)
                            for ex in range(-_R, _R + 1)
                            if ey * ey + ex * ex == d2)
            d = dk_ref[_R - rey, _R - rex]
            m = mask_ref[_R - rey, _R - rex]
            w_pad = jnp.clip(prad - (d - 1.0), 0.0, 1.0) * m
            p = [w_pad * pad_ref[c, y0:y0 + _TH + 2 * _R, :] for c in range(3)]
            p.append(w_pad)
            for ady, dxs in groups:
                for ci in range(4):
                    xs = None
                    for dx in dxs:
                        t = p[ci][:, _R + dx:_R + dx + _W]
                        xs = t if xs is None else xs + t
                    for ey in ({0} if ady == 0 else {-ady, ady}):
                        t = xs[_R + ey:_R + ey + _TH, :]
                        acc[ci] = t if acc[ci] is None else acc[ci] + t

        inv = 1.0 / (acc[3] + 1e-8)
        o_ref[0, 0, y0:y0 + _TH, :] = acc[0] * inv
        o_ref[0, 1, y0:y0 + _TH, :] = acc[1] * inv
        o_ref[0, 2, y0:y0 + _TH, :] = acc[2] * inv


@jax.jit
def kernel(x, lens_effects, diskernel, lens_mask):
    b, c, h, w = x.shape
    out = pl.pallas_call(
        _scatter_body,
        grid=(b,),
        in_specs=[
            pl.BlockSpec((b, 1), lambda i: (0, 0), memory_space=pltpu.SMEM),
            pl.BlockSpec((_L, _L), lambda i: (0, 0), memory_space=pltpu.SMEM),
            pl.BlockSpec((_L, _L), lambda i: (0, 0), memory_space=pltpu.SMEM),
            pl.BlockSpec((1, 4, h, w), lambda i: (i, 0, 0, 0)),
        ],
        out_specs=pl.BlockSpec((1, 3, h, w), lambda i: (i, 0, 0, 0)),
        out_shape=jax.ShapeDtypeStruct((b, 3, h, w), x.dtype),
        scratch_shapes=[pltpu.VMEM((4, h + 2 * _R, w + 2 * _R), jnp.float32)],
    )(lens_effects, diskernel, lens_mask, x)
    return out
